# Initial kernel scaffold; baseline (speedup 1.0000x reference)
import functools

import jax
import jax.numpy as jnp
from jax import lax
from jax.experimental import pallas as pl
from jax.experimental.pallas import tpu as pltpu
from jax.experimental.pallas import tpu_sc as plsc

TOPK = 64
VOCAB = 1_000_000
CHUNK = 20_000
NCHUNK = VOCAB // CHUNK
BLK = 2_000
BPC = CHUNK // BLK
NBLK = VOCAB // BLK
NBLK_PAD = 512
VPB = BLK // 16
CAP = 2_048
NROW = 32
OUTW = 16

NEG_INF = jnp.float32(-jnp.inf)
IMAX = jnp.int32(2**31 - 1)


def _fold_key(m):
  return lax.bitwise_xor(
      m, lax.bitwise_and(lax.shift_right_arithmetic(m, 31),
                         jnp.int32(0x7FFFFFFF)))


def _row_body(lp_hbm, g_hbm, outs_hbm, outi_hbm,
              buf0, buf1, bbuf, bmax, bkey, cand_v, cand_i,
              topv, topi, gbuf, ovec, oivec, cc_ref, sem0, sem1):
  r = lax.axis_index("c") * 16 + lax.axis_index("s")
  iota = lax.broadcasted_iota(jnp.int32, (16,), 0)
  ninf_v = jnp.full((16,), NEG_INF, jnp.float32)

  bufs = (buf0, buf1)
  sems = (sem0, sem1)

  def chunk_copy(c, b):
    return pltpu.make_async_copy(
        lp_hbm.at[r, 0, pl.ds(c * CHUNK, CHUNK)], bufs[b], sems[b])

  chunk_copy(0, 0).start()
  chunk_copy(1, 1).start()

  def fill_body(i, _):
    cand_v[pl.ds(i * 16, 16)] = ninf_v
    cand_i[pl.ds(i * 16, 16)] = jnp.full((16,), IMAX, jnp.int32)
    return 0
  lax.fori_loop(0, CAP // 16, fill_body, 0)
  for j in range(NBLK, NBLK_PAD):
    bmax[j] = NEG_INF

  def do_chunk(c, buf):
    def blk_body(j, _):
      base = j * BLK
      accs = [ninf_v] * 5
      for i in range(VPB):
        q = i % 5
        accs[q] = jnp.maximum(accs[q], buf[pl.ds(base + i * 16, 16)])
      acc = jnp.maximum(jnp.maximum(jnp.maximum(accs[0], accs[1]),
                                    jnp.maximum(accs[2], accs[3])), accs[4])
      bmax[c * BPC + j] = jnp.max(acc)
      return 0
    lax.fori_loop(0, BPC, blk_body, 0)

  def pair_body(k, _):
    for b in range(2):
      c = 2 * k + b
      chunk_copy(c, b).wait()
      do_chunk(c, bufs[b])

      @pl.when(c + 2 < NCHUNK)
      def _start_next():
        chunk_copy(c + 2, b).start()
    return 0
  lax.fori_loop(0, NCHUNK // 2, pair_body, 0)

  def key_body(i, _):
    m = plsc.bitcast(bmax[pl.ds(i * 16, 16)], jnp.int32)
    bkey[pl.ds(i * 16, 16)] = _fold_key(m)
    return 0
  lax.fori_loop(0, NBLK_PAD // 16, key_body, 0)

  def bit_body(b, prefix):
    cand = lax.bitwise_or(prefix, lax.shift_left(jnp.int32(1), 30 - b))
    def cnt_body(i, acc):
      kv = bkey[pl.ds(i * 16, 16)]
      return acc + jnp.where(kv >= cand, jnp.int32(1), jnp.int32(0))
    acc = lax.fori_loop(0, NBLK_PAD // 16, cnt_body,
                        jnp.zeros((16,), jnp.int32))
    return jnp.where(jnp.sum(acc) >= TOPK, cand, prefix)
  v64k = lax.fori_loop(0, 31, bit_body, jnp.int32(-2**31))
  t = lax.bitcast_convert_type(_fold_key(v64k), jnp.float32)

  cc_ref[0] = jnp.int32(0)

  def blk_scan(b, _):
    @pl.when(bmax[b] >= t)
    def _scan_block():
      pltpu.sync_copy(lp_hbm.at[r, 0, pl.ds(b * BLK, BLK)], bbuf)
      def v_body(i, _):
        x = bbuf[pl.ds(i * 16, 16)]
        msk = x >= t
        @pl.when(jnp.any(msk))
        def _append():
          mi = jnp.where(msk, jnp.int32(1), jnp.int32(0))
          cc = cc_ref[0]
          @pl.when(cc < CAP - 16)
          def _store():
            pos = plsc.cumsum(mi) + (cc - 1)
            plsc.store_scatter(cand_v, [pos], x, mask=msk)
            plsc.store_scatter(cand_i, [pos], b * BLK + i * 16 + iota,
                               mask=msk)
          cc_ref[0] = cc + jnp.sum(mi)
        return 0
      lax.fori_loop(0, VPB, v_body, 0)
    return 0
  lax.fori_loop(0, NBLK, blk_scan, 0)

  nv = (jnp.minimum(cc_ref[0], CAP) + 15) // 16

  def round_body(k, _):
    def mx(i, acc):
      return jnp.maximum(acc, cand_v[pl.ds(i * 16, 16)])
    m = jnp.max(lax.fori_loop(0, nv, mx, ninf_v))
    def mi(i, acc):
      hit = cand_v[pl.ds(i * 16, 16)] == m
      return jnp.minimum(acc, jnp.where(hit, cand_i[pl.ds(i * 16, 16)], IMAX))
    j = jnp.min(lax.fori_loop(0, nv, mi, jnp.full((16,), IMAX, jnp.int32)))
    def cl(i, _):
      hit = (cand_v[pl.ds(i * 16, 16)] == m) & (cand_i[pl.ds(i * 16, 16)] == j)
      cand_v[pl.ds(i * 16, 16)] = jnp.where(hit, NEG_INF,
                                            cand_v[pl.ds(i * 16, 16)])
      return 0
    lax.fori_loop(0, nv, cl, 0)
    topv[k] = m
    topi[k] = j
    return 0
  lax.fori_loop(0, TOPK, round_body, 0)

  pltpu.sync_copy(g_hbm.at[r], gbuf)
  ovec[pl.ds(0, 16)] = jnp.zeros((16,), jnp.float32)
  oivec[pl.ds(0, 16)] = jnp.zeros((16,), jnp.int32)
  for d in range(2):
    svecs = [topv[pl.ds(q * 16, 16)] + gbuf[pl.ds(d * TOPK + q * 16, 16)]
             for q in range(TOPK // 16)]
    m = jnp.max(jnp.maximum(jnp.maximum(svecs[0], svecs[1]),
                            jnp.maximum(svecs[2], svecs[3])))
    kacc = jnp.full((16,), IMAX, jnp.int32)
    for q in range(TOPK // 16):
      kacc = jnp.minimum(kacc, jnp.where(svecs[q] == m, iota + q * 16, IMAX))
    p = jnp.min(kacc)
    ovec[d] = topv[p]
    oivec[d] = topi[p]

  pltpu.sync_copy(ovec, outs_hbm.at[r])
  pltpu.sync_copy(oivec, outi_hbm.at[r])


@jax.jit
def _sc_topk_sample(lprobs, gmat):
  mesh = plsc.VectorSubcoreMesh(core_axis_name="c", subcore_axis_name="s")
  fn = pl.kernel(
      _row_body,
      out_type=[
          jax.ShapeDtypeStruct((NROW, OUTW), jnp.float32),
          jax.ShapeDtypeStruct((NROW, OUTW), jnp.int32),
      ],
      mesh=mesh,
      scratch_types=[
          pltpu.VMEM((CHUNK,), jnp.float32),
          pltpu.VMEM((CHUNK,), jnp.float32),
          pltpu.VMEM((BLK,), jnp.float32),
          pltpu.VMEM((NBLK_PAD,), jnp.float32),
          pltpu.VMEM((NBLK_PAD,), jnp.int32),
          pltpu.VMEM((CAP,), jnp.float32),
          pltpu.VMEM((CAP,), jnp.int32),
          pltpu.VMEM((TOPK,), jnp.float32),
          pltpu.VMEM((TOPK,), jnp.int32),
          pltpu.VMEM((2 * TOPK,), jnp.float32),
          pltpu.VMEM((OUTW,), jnp.float32),
          pltpu.VMEM((OUTW,), jnp.int32),
          pltpu.SMEM((1,), jnp.int32),
          pltpu.SemaphoreType.DMA,
          pltpu.SemaphoreType.DMA,
      ],
  )
  return fn(lprobs, gmat)


def kernel(step, lprobs, scores):
  bsz, beam_size, _ = lprobs.shape
  g = jax.random.gumbel(jax.random.key(42), (beam_size, bsz, TOPK),
                        jnp.float32)
  gmat = jnp.transpose(g, (1, 0, 2)).reshape(bsz, beam_size * TOPK)
  outs, outi = _sc_topk_sample(lprobs, gmat)
  scores_buf = outs[:, :beam_size] + scores[:, :, 0] * step
  indices_buf = outi[:, :beam_size]
  beams_buf = jnp.full((bsz, beam_size), step, dtype=indices_buf.dtype)
  return scores_buf, indices_buf, beams_buf


# speedup vs baseline: 53.5153x; 53.5153x over previous
import jax
import jax.numpy as jnp
import numpy as np
from jax import lax
from jax.experimental import pallas as pl
from jax.experimental.pallas import tpu as pltpu
from jax.experimental.pallas import tpu_sc as plsc

TOPK = 64
VOCAB = 1_000_000
ROW_STRIDE = 2 * VOCAB
CHUNK = 16_384
NCHUNK = 61
BLK = 2_048
BPC = CHUNK // BLK
NBLK_FULL = 488
TAIL_OFF = NBLK_FULL * BLK
TAIL_VREGS = 36
TAIL_FETCH = 1_024
NBLK = NBLK_FULL + 1
NLM = NBLK * 16
NLM_PAD = 8_192
VPB = BLK // 16
CAP = 2_048
NROW = 32
OUTW = 128

NEG_INF = np.float32(-np.inf)
IMAX = np.int32(2**31 - 1)


def _fold_key(m):
  return lax.bitwise_xor(
      m, lax.bitwise_and(lax.shift_right_arithmetic(m, 31),
                         np.int32(0x7FFFFFFF)))


def _row_body(lp_hbm, g_hbm, outs_hbm, outi_hbm,
              buf0, buf1, bbuf, bmax, bkey, cand_v, cand_i,
              topv, topi, gbuf, ovec, oivec, cc_ref, sem0, sem1):
  r = lax.axis_index("c") * 16 + lax.axis_index("s")
  rbase = r * ROW_STRIDE
  iota = lax.broadcasted_iota(jnp.int32, (16,), 0)
  ninf_v = jnp.full((16,), NEG_INF, jnp.float32)
  lane0 = iota == 0

  bufs = (buf0, buf1)
  sems = (sem0, sem1)

  def chunk_copy(c, b):
    return pltpu.make_async_copy(
        lp_hbm.at[pl.ds(rbase + c * CHUNK, CHUNK)], bufs[b], sems[b])

  chunk_copy(0, 0).start()
  chunk_copy(1, 1).start()

  def fill_body(i, _):
    cand_v[pl.ds(i * 16, 16)] = ninf_v
    cand_i[pl.ds(i * 16, 16)] = jnp.full((16,), IMAX, jnp.int32)
    return 0
  lax.fori_loop(0, CAP // 16, fill_body, 0)
  for j in range(NLM // 16, NLM_PAD // 16):
    bmax[pl.ds(j * 16, 16)] = ninf_v

  def do_chunk(c, buf):
    def blk_body(j, _):
      base = j * BLK
      accs = [ninf_v] * 4
      for i in range(VPB):
        q = i % 4
        accs[q] = jnp.maximum(accs[q], buf[pl.ds(base + i * 16, 16)])
      acc = jnp.maximum(jnp.maximum(accs[0], accs[1]),
                        jnp.maximum(accs[2], accs[3]))
      bmax[pl.ds((c * BPC + j) * 16, 16)] = acc
      return 0
    lax.fori_loop(0, BPC, blk_body, 0)

  def pair_body(k, _):
    for b in range(2):
      c = 2 * k + b
      chunk_copy(c, b).wait()
      do_chunk(c, bufs[b])

      @pl.when(c + 2 < NCHUNK)
      def _start_next():
        chunk_copy(c + 2, b).start()
    return 0
  lax.fori_loop(0, (NCHUNK - 1) // 2, pair_body, 0)

  chunk_copy(NCHUNK - 1, 0).wait()
  tail_cp = pltpu.make_async_copy(
      lp_hbm.at[pl.ds(rbase + TAIL_OFF, TAIL_FETCH)],
      buf1.at[pl.ds(0, TAIL_FETCH)], sem1)
  tail_cp.start()
  do_chunk(NCHUNK - 1, buf0)
  tail_cp.wait()
  tacc = ninf_v
  for i in range(TAIL_VREGS):
    tacc = jnp.maximum(tacc, buf1[pl.ds(i * 16, 16)])
  bmax[pl.ds(NBLK_FULL * 16, 16)] = tacc

  def key_body(i, _):
    m = lax.bitcast_convert_type(bmax[pl.ds(i * 16, 16)], jnp.int32)
    bkey[pl.ds(i * 16, 16)] = _fold_key(m)
    return 0
  lax.fori_loop(0, NLM_PAD // 16, key_body, 0)

  def bit_body(b, prefix_u):
    cand_u = lax.bitwise_or(prefix_u, lax.shift_left(np.int32(1), 31 - b))
    cand_s = lax.bitwise_xor(cand_u, np.int32(-2**31))
    def cnt_body(i, acc):
      kv = bkey[pl.ds(i * 16, 16)]
      return acc + jnp.where(kv >= cand_s, np.int32(1), np.int32(0))
    acc = lax.fori_loop(0, NLM_PAD // 16, cnt_body,
                        jnp.zeros((16,), jnp.int32))
    return jnp.where(jnp.sum(acc) >= TOPK, cand_u, prefix_u)
  v64u = lax.fori_loop(0, 32, bit_body, np.int32(0))
  v64k = lax.bitwise_xor(v64u, np.int32(-2**31))
  t = lax.bitcast_convert_type(_fold_key(v64k), jnp.float32)

  cc_ref[0] = np.int32(0)

  def append_vreg(x, gbase):
    msk = x >= t
    @pl.when(jnp.any(msk))
    def _append():
      mi = jnp.where(msk, np.int32(1), np.int32(0))
      cc = cc_ref[0]
      @pl.when(cc < CAP - 16)
      def _store():
        pos = plsc.cumsum(mi) + (cc - 1)
        plsc.store_scatter(cand_v, [pos], x, mask=msk)
        plsc.store_scatter(cand_i, [pos], gbase + iota, mask=msk)
      cc_ref[0] = cc + jnp.sum(mi)

  def blk_scan(b, _):
    @pl.when(jnp.any(bmax[pl.ds(b * 16, 16)] >= t))
    def _scan_block():
      pltpu.sync_copy(lp_hbm.at[pl.ds(rbase + b * BLK, BLK)], bbuf)
      def v_body(i, _):
        append_vreg(bbuf[pl.ds(i * 16, 16)], b * BLK + i * 16)
        return 0
      lax.fori_loop(0, VPB, v_body, 0)
    return 0
  lax.fori_loop(0, NBLK_FULL, blk_scan, 0)

  @pl.when(jnp.any(bmax[pl.ds(NBLK_FULL * 16, 16)] >= t))
  def _scan_tail():
    pltpu.sync_copy(lp_hbm.at[pl.ds(rbase + TAIL_OFF, TAIL_FETCH)],
                    bbuf.at[pl.ds(0, TAIL_FETCH)])
    def v_body(i, _):
      append_vreg(bbuf[pl.ds(i * 16, 16)], TAIL_OFF + i * 16)
      return 0
    lax.fori_loop(0, TAIL_VREGS, v_body, 0)

  nv = (jnp.minimum(cc_ref[0], CAP) + 15) // 16

  def round_body(k, _):
    def mx(i, acc):
      return jnp.maximum(acc, cand_v[pl.ds(i * 16, 16)])
    m = jnp.max(lax.fori_loop(0, nv, mx, ninf_v))
    def mi(i, acc):
      hit = cand_v[pl.ds(i * 16, 16)] == m
      return jnp.minimum(acc, jnp.where(hit, cand_i[pl.ds(i * 16, 16)], IMAX))
    j = jnp.min(lax.fori_loop(0, nv, mi, jnp.full((16,), IMAX, jnp.int32)))
    def cl(i, _):
      hit = (cand_v[pl.ds(i * 16, 16)] == m) & (cand_i[pl.ds(i * 16, 16)] == j)
      cand_v[pl.ds(i * 16, 16)] = jnp.where(hit, NEG_INF,
                                            cand_v[pl.ds(i * 16, 16)])
      return 0
    lax.fori_loop(0, nv, cl, 0)
    kpos = jnp.broadcast_to(k, (16,)).astype(jnp.int32)
    plsc.store_scatter(topv, [kpos], jnp.broadcast_to(m, (16,)), mask=lane0)
    plsc.store_scatter(topi, [kpos], jnp.broadcast_to(j, (16,)), mask=lane0)
    return 0
  lax.fori_loop(0, TOPK, round_body, 0)

  pltpu.sync_copy(g_hbm.at[pl.ds(r * OUTW, OUTW)], gbuf)
  for i in range(OUTW // 16):
    ovec[pl.ds(i * 16, 16)] = jnp.zeros((16,), jnp.float32)
    oivec[pl.ds(i * 16, 16)] = jnp.zeros((16,), jnp.int32)
  for d in range(2):
    svecs = [topv[pl.ds(q * 16, 16)] + gbuf[pl.ds(d * TOPK + q * 16, 16)]
             for q in range(TOPK // 16)]
    m = jnp.max(jnp.maximum(jnp.maximum(svecs[0], svecs[1]),
                            jnp.maximum(svecs[2], svecs[3])))
    kacc = jnp.full((16,), IMAX, jnp.int32)
    for q in range(TOPK // 16):
      kacc = jnp.minimum(kacc, jnp.where(svecs[q] == m, iota + q * 16, IMAX))
    p = jnp.min(kacc)
    sel = jnp.broadcast_to(p, (16,))
    dpos = jnp.full((16,), d, jnp.int32)
    plsc.store_scatter(ovec, [dpos], plsc.load_gather(topv, [sel]),
                       mask=lane0)
    plsc.store_scatter(oivec, [dpos], plsc.load_gather(topi, [sel]),
                       mask=lane0)

  pltpu.sync_copy(ovec, outs_hbm.at[pl.ds(r * OUTW, OUTW)])
  pltpu.sync_copy(oivec, outi_hbm.at[pl.ds(r * OUTW, OUTW)])


@jax.jit
def _sc_topk_sample(lp_flat, g_flat):
  mesh = plsc.VectorSubcoreMesh(core_axis_name="c", subcore_axis_name="s")
  fn = pl.kernel(
      _row_body,
      out_type=[
          jax.ShapeDtypeStruct((NROW * OUTW,), jnp.float32),
          jax.ShapeDtypeStruct((NROW * OUTW,), jnp.int32),
      ],
      mesh=mesh,
      compiler_params=pltpu.CompilerParams(needs_layout_passes=False),
      scratch_types=[
          pltpu.VMEM((CHUNK,), jnp.float32),
          pltpu.VMEM((CHUNK,), jnp.float32),
          pltpu.VMEM((BLK,), jnp.float32),
          pltpu.VMEM((NLM_PAD,), jnp.float32),
          pltpu.VMEM((NLM_PAD,), jnp.int32),
          pltpu.VMEM((CAP,), jnp.float32),
          pltpu.VMEM((CAP,), jnp.int32),
          pltpu.VMEM((TOPK,), jnp.float32),
          pltpu.VMEM((TOPK,), jnp.int32),
          pltpu.VMEM((2 * TOPK,), jnp.float32),
          pltpu.VMEM((OUTW,), jnp.float32),
          pltpu.VMEM((OUTW,), jnp.int32),
          pltpu.SMEM((1,), jnp.int32),
          pltpu.SemaphoreType.DMA,
          pltpu.SemaphoreType.DMA,
      ],
  )
  return fn(lp_flat, g_flat)


def kernel(step, lprobs, scores):
  bsz, beam_size, _ = lprobs.shape
  g = jax.random.gumbel(jax.random.key(42), (beam_size, bsz, TOPK),
                        jnp.float32)
  gmat = jnp.zeros((bsz, OUTW), jnp.float32)
  gmat = gmat.at[:, :beam_size * TOPK].set(
      jnp.transpose(g, (1, 0, 2)).reshape(bsz, beam_size * TOPK))
  outs, outi = _sc_topk_sample(lprobs.reshape(-1), gmat.reshape(-1))
  outs = outs.reshape(NROW, OUTW)
  outi = outi.reshape(NROW, OUTW)
  scores_buf = outs[:, :beam_size] + scores[:, :, 0] * step
  indices_buf = outi[:, :beam_size]
  beams_buf = jnp.full((bsz, beam_size), step, dtype=indices_buf.dtype)
  return scores_buf, indices_buf, beams_buf


# speedup vs baseline: 354.0075x; 6.6151x over previous
import jax
import jax.numpy as jnp
import numpy as np
from jax import lax
from jax.experimental import pallas as pl
from jax.experimental.pallas import tpu as pltpu
from jax.experimental.pallas import tpu_sc as plsc

TOPK = 64
VOCAB = 1_000_000
CHUNK = 16_384
NCHUNK = 61
BLK = 2_048
BPC = CHUNK // BLK
NBLK_FULL = 488
TAIL_OFF = NBLK_FULL * BLK
TAIL_VREGS = 36
TAIL_W = 640
NBLK = NBLK_FULL + 1
NBLK_PAD = 512
NLM = NBLK * 16
NLM_PAD = 8_192
VPB = BLK // 16
CAP = 2_048
NROW = 32
OUTW = 128

NEG_INF = np.float32(-np.inf)
IMAX = np.int32(2**31 - 1)


def _fold_key(m):
  return lax.bitwise_xor(
      m, lax.bitwise_and(lax.shift_right_arithmetic(m, 31),
                         np.int32(0x7FFFFFFF)))


def _row_body(lp_hbm, tail_hbm, g_hbm, outs_hbm, outi_hbm,
              buf0, buf1, bbuf, tbuf, bmax, bblk, bkey, cand_v, cand_i,
              topv, topi, gbuf, ovec, oivec, cc_ref, sem0, sem1):
  r = lax.axis_index("c") * 16 + lax.axis_index("s")
  iota = lax.broadcasted_iota(jnp.int32, (16,), 0)
  ninf_v = jnp.full((16,), NEG_INF, jnp.float32)
  lane0 = iota == 0

  bufs = (buf0, buf1)
  sems = (sem0, sem1)

  def chunk_copy(c, b):
    return pltpu.make_async_copy(
        lp_hbm.at[pl.ds(2 * r, 2), pl.ds(c * CHUNK, CHUNK)],
        bufs[b], sems[b])

  chunk_copy(0, 0).start()
  chunk_copy(1, 1).start()

  def fill_body(i, _):
    cand_v[pl.ds(i * 16, 16)] = ninf_v
    cand_i[pl.ds(i * 16, 16)] = jnp.full((16,), IMAX, jnp.int32)
    return 0
  lax.fori_loop(0, CAP // 16, fill_body, 0)
  for j in range(NLM // 16, NLM_PAD // 16):
    bmax[pl.ds(j * 16, 16)] = ninf_v
  for j in range(NBLK // 16, NBLK_PAD // 16):
    bblk[pl.ds(j * 16, 16)] = ninf_v

  def do_chunk(c, buf):
    def blk_body(j, _):
      base = j * BLK
      accs = [ninf_v] * 4
      for i in range(VPB):
        q = i % 4
        accs[q] = jnp.maximum(accs[q], buf[0, pl.ds(base + i * 16, 16)])
      acc = jnp.maximum(jnp.maximum(accs[0], accs[1]),
                        jnp.maximum(accs[2], accs[3]))
      blkid = c * BPC + j
      bmax[pl.ds(blkid * 16, 16)] = acc
      bpos = jnp.broadcast_to(blkid, (16,)).astype(jnp.int32)
      plsc.store_scatter(bblk, [bpos],
                         jnp.broadcast_to(jnp.max(acc), (16,)), mask=lane0)
      return 0
    lax.fori_loop(0, BPC, blk_body, 0)

  def pair_body(k, _):
    for b in range(2):
      c = 2 * k + b
      chunk_copy(c, b).wait()
      do_chunk(c, bufs[b])

      @pl.when(c + 2 < NCHUNK)
      def _start_next():
        chunk_copy(c + 2, b).start()
    return 0
  lax.fori_loop(0, (NCHUNK - 1) // 2, pair_body, 0)

  chunk_copy(NCHUNK - 1, 0).wait()
  tail_cp = pltpu.make_async_copy(
      tail_hbm.at[pl.ds(r * TAIL_W, TAIL_W)], tbuf, sem1)
  tail_cp.start()
  do_chunk(NCHUNK - 1, buf0)
  tail_cp.wait()
  tacc = ninf_v
  for i in range(TAIL_VREGS):
    tacc = jnp.maximum(tacc, tbuf[pl.ds(i * 16, 16)])
  bmax[pl.ds(NBLK_FULL * 16, 16)] = tacc
  tpos = jnp.full((16,), NBLK_FULL, jnp.int32)
  plsc.store_scatter(bblk, [tpos],
                     jnp.broadcast_to(jnp.max(tacc), (16,)), mask=lane0)

  def key_body(i, _):
    m = lax.bitcast_convert_type(bblk[pl.ds(i * 16, 16)], jnp.int32)
    bkey[pl.ds(i * 16, 16)] = _fold_key(m)
    return 0
  lax.fori_loop(0, NBLK_PAD // 16, key_body, 0)

  def bit_body(b, prefix_u):
    cand_u = lax.bitwise_or(prefix_u, lax.shift_left(np.int32(1), 31 - b))
    cand_s = lax.bitwise_xor(cand_u, np.int32(-2**31))
    def cnt_body(i, acc):
      kv = bkey[pl.ds(i * 16, 16)]
      return acc + jnp.where(kv >= cand_s, np.int32(1), np.int32(0))
    acc = lax.fori_loop(0, NBLK_PAD // 16, cnt_body,
                        jnp.zeros((16,), jnp.int32))
    return jnp.where(jnp.sum(acc) >= TOPK, cand_u, prefix_u)
  v64u = lax.fori_loop(0, 32, bit_body, np.int32(0))
  v64k = lax.bitwise_xor(v64u, np.int32(-2**31))
  t = lax.bitcast_convert_type(_fold_key(v64k), jnp.float32)

  cc_ref[0] = np.int32(0)

  def append_vreg(x, gbase):
    msk = x >= t
    @pl.when(jnp.any(msk))
    def _append():
      mi = jnp.where(msk, np.int32(1), np.int32(0))
      cc = cc_ref[0]
      @pl.when(cc < CAP - 16)
      def _store():
        pos = plsc.cumsum(mi) + (cc - 1)
        plsc.store_scatter(cand_v, [pos], x, mask=msk)
        plsc.store_scatter(cand_i, [pos], gbase + iota, mask=msk)
      cc_ref[0] = cc + jnp.sum(mi)

  def blk_scan(b, _):
    @pl.when(jnp.any(bmax[pl.ds(b * 16, 16)] >= t))
    def _scan_block():
      pltpu.sync_copy(lp_hbm.at[pl.ds(2 * r, 2), pl.ds(b * BLK, BLK)], bbuf)
      def v_body(i, _):
        append_vreg(bbuf[0, pl.ds(i * 16, 16)], b * BLK + i * 16)
        return 0
      lax.fori_loop(0, VPB, v_body, 0)
    return 0
  lax.fori_loop(0, NBLK_FULL, blk_scan, 0)

  @pl.when(jnp.any(bmax[pl.ds(NBLK_FULL * 16, 16)] >= t))
  def _scan_tail():
    pltpu.sync_copy(tail_hbm.at[pl.ds(r * TAIL_W, TAIL_W)], tbuf)
    def v_body(i, _):
      append_vreg(tbuf[pl.ds(i * 16, 16)], TAIL_OFF + i * 16)
      return 0
    lax.fori_loop(0, TAIL_VREGS, v_body, 0)

  nv = (jnp.minimum(cc_ref[0], CAP) + 15) // 16

  def round_body(k, _):
    def mx(i, acc):
      return jnp.maximum(acc, cand_v[pl.ds(i * 16, 16)])
    m = jnp.max(lax.fori_loop(0, nv, mx, ninf_v))
    def mi(i, acc):
      hit = cand_v[pl.ds(i * 16, 16)] == m
      return jnp.minimum(acc, jnp.where(hit, cand_i[pl.ds(i * 16, 16)], IMAX))
    j = jnp.min(lax.fori_loop(0, nv, mi, jnp.full((16,), IMAX, jnp.int32)))
    def cl(i, _):
      hit = (cand_v[pl.ds(i * 16, 16)] == m) & (cand_i[pl.ds(i * 16, 16)] == j)
      cand_v[pl.ds(i * 16, 16)] = jnp.where(hit, NEG_INF,
                                            cand_v[pl.ds(i * 16, 16)])
      return 0
    lax.fori_loop(0, nv, cl, 0)
    kpos = jnp.broadcast_to(k, (16,)).astype(jnp.int32)
    plsc.store_scatter(topv, [kpos], jnp.broadcast_to(m, (16,)), mask=lane0)
    plsc.store_scatter(topi, [kpos], jnp.broadcast_to(j, (16,)), mask=lane0)
    return 0
  lax.fori_loop(0, TOPK, round_body, 0)

  pltpu.sync_copy(g_hbm.at[pl.ds(r * OUTW, OUTW)], gbuf)
  for i in range(OUTW // 16):
    ovec[pl.ds(i * 16, 16)] = jnp.zeros((16,), jnp.float32)
    oivec[pl.ds(i * 16, 16)] = jnp.zeros((16,), jnp.int32)
  for d in range(2):
    svecs = [topv[pl.ds(q * 16, 16)] + gbuf[pl.ds(d * TOPK + q * 16, 16)]
             for q in range(TOPK // 16)]
    m = jnp.max(jnp.maximum(jnp.maximum(svecs[0], svecs[1]),
                            jnp.maximum(svecs[2], svecs[3])))
    kacc = jnp.full((16,), IMAX, jnp.int32)
    for q in range(TOPK // 16):
      kacc = jnp.minimum(kacc, jnp.where(svecs[q] == m, iota + q * 16, IMAX))
    p = jnp.min(kacc)
    sel = jnp.broadcast_to(p, (16,))
    dpos = jnp.full((16,), d, jnp.int32)
    plsc.store_scatter(ovec, [dpos], plsc.load_gather(topv, [sel]),
                       mask=lane0)
    plsc.store_scatter(oivec, [dpos], plsc.load_gather(topi, [sel]),
                       mask=lane0)

  pltpu.sync_copy(ovec, outs_hbm.at[pl.ds(r * OUTW, OUTW)])
  pltpu.sync_copy(oivec, outi_hbm.at[pl.ds(r * OUTW, OUTW)])


@jax.jit
def _sc_topk_sample(lp2, tail_flat, g_flat):
  mesh = plsc.VectorSubcoreMesh(core_axis_name="c", subcore_axis_name="s")
  fn = pl.kernel(
      _row_body,
      out_type=[
          jax.ShapeDtypeStruct((NROW * OUTW,), jnp.float32),
          jax.ShapeDtypeStruct((NROW * OUTW,), jnp.int32),
      ],
      mesh=mesh,
      compiler_params=pltpu.CompilerParams(needs_layout_passes=False),
      scratch_types=[
          pltpu.VMEM((2, CHUNK), jnp.float32),
          pltpu.VMEM((2, CHUNK), jnp.float32),
          pltpu.VMEM((2, BLK), jnp.float32),
          pltpu.VMEM((TAIL_W,), jnp.float32),
          pltpu.VMEM((NLM_PAD,), jnp.float32),
          pltpu.VMEM((NBLK_PAD,), jnp.float32),
          pltpu.VMEM((NBLK_PAD,), jnp.int32),
          pltpu.VMEM((CAP,), jnp.float32),
          pltpu.VMEM((CAP,), jnp.int32),
          pltpu.VMEM((TOPK,), jnp.float32),
          pltpu.VMEM((TOPK,), jnp.int32),
          pltpu.VMEM((2 * TOPK,), jnp.float32),
          pltpu.VMEM((OUTW,), jnp.float32),
          pltpu.VMEM((OUTW,), jnp.int32),
          pltpu.SMEM((1,), jnp.int32),
          pltpu.SemaphoreType.DMA,
          pltpu.SemaphoreType.DMA,
      ],
  )
  return fn(lp2, tail_flat, g_flat)


def kernel(step, lprobs, scores):
  bsz, beam_size, _ = lprobs.shape
  g = jax.random.gumbel(jax.random.key(42), (beam_size, bsz, TOPK),
                        jnp.float32)
  gmat = jnp.zeros((bsz, OUTW), jnp.float32)
  gmat = gmat.at[:, :beam_size * TOPK].set(
      jnp.transpose(g, (1, 0, 2)).reshape(bsz, beam_size * TOPK))
  lp2 = lprobs.reshape(bsz * beam_size, VOCAB)
  tail_p = jnp.full((bsz, TAIL_W), NEG_INF, jnp.float32)
  tail_p = tail_p.at[:, :VOCAB - TAIL_OFF].set(lprobs[:, 0, TAIL_OFF:])
  outs, outi = _sc_topk_sample(lp2, tail_p.reshape(-1), gmat.reshape(-1))
  outs = outs.reshape(NROW, OUTW)
  outi = outi.reshape(NROW, OUTW)
  scores_buf = outs[:, :beam_size] + scores[:, :, 0] * step
  indices_buf = outi[:, :beam_size]
  beams_buf = jnp.full((bsz, beam_size), step, dtype=indices_buf.dtype)
  return scores_buf, indices_buf, beams_buf


# speedup vs baseline: 380.4725x; 1.0748x over previous
import jax
import jax.numpy as jnp
import numpy as np
from jax import lax
from jax.experimental import pallas as pl
from jax.experimental.pallas import tpu as pltpu
from jax.experimental.pallas import tpu_sc as plsc

TOPK = 64
VOCAB = 1_000_000
CHUNK = 16_384
NCHUNK = 61
BLK = 2_048
BPC = CHUNK // BLK
NBLK_FULL = 488
TAIL_OFF = NBLK_FULL * BLK
TAIL_VREGS = 36
TAIL_W = 640
NBLK = NBLK_FULL + 1
NBLK_PAD = 512
NLM = NBLK * 16
NLM_PAD = 8_192
VPB = BLK // 16
CAP = 2_048
NROW = 32
OUTW = 128

NEG_INF = np.float32(-np.inf)
IMAX = np.int32(2**31 - 1)


def _fold_key(m):
  return lax.bitwise_xor(
      m, lax.bitwise_and(lax.shift_right_arithmetic(m, 31),
                         np.int32(0x7FFFFFFF)))


def _row_body(lp_hbm, tail_hbm, g_hbm, outs_hbm, outi_hbm,
              buf0, buf1, bbuf, bbuf2, tbuf, bmax, bblk, bkey, qlist,
              cand_v, cand_i,
              topv, topi, gbuf, ovec, oivec, cc_ref, sem0, sem1):
  r = lax.axis_index("c") * 16 + lax.axis_index("s")
  iota = lax.broadcasted_iota(jnp.int32, (16,), 0)
  ninf_v = jnp.full((16,), NEG_INF, jnp.float32)
  lane0 = iota == 0

  bufs = (buf0, buf1)
  sems = (sem0, sem1)

  def chunk_copy(c, b):
    return pltpu.make_async_copy(
        lp_hbm.at[pl.ds(2 * r, 2), pl.ds(c * CHUNK, CHUNK)],
        bufs[b], sems[b])

  chunk_copy(0, 0).start()
  chunk_copy(1, 1).start()

  def fill_body(i, _):
    cand_v[pl.ds(i * 16, 16)] = ninf_v
    cand_i[pl.ds(i * 16, 16)] = jnp.full((16,), IMAX, jnp.int32)
    return 0
  lax.fori_loop(0, CAP // 16, fill_body, 0)
  for j in range(NLM // 16, NLM_PAD // 16):
    bmax[pl.ds(j * 16, 16)] = ninf_v
  for j in range(NBLK // 16, NBLK_PAD // 16):
    bblk[pl.ds(j * 16, 16)] = ninf_v

  def do_chunk(c, buf):
    def blk_body(j, _):
      base = j * BLK
      accs = [ninf_v] * 4
      for i in range(VPB):
        q = i % 4
        accs[q] = jnp.maximum(accs[q], buf[0, pl.ds(base + i * 16, 16)])
      acc = jnp.maximum(jnp.maximum(accs[0], accs[1]),
                        jnp.maximum(accs[2], accs[3]))
      blkid = c * BPC + j
      bmax[pl.ds(blkid * 16, 16)] = acc
      bpos = jnp.broadcast_to(blkid, (16,)).astype(jnp.int32)
      plsc.store_scatter(bblk, [bpos],
                         jnp.broadcast_to(jnp.max(acc), (16,)), mask=lane0)
      return 0
    lax.fori_loop(0, BPC, blk_body, 0)

  def pair_body(k, _):
    for b in range(2):
      c = 2 * k + b
      chunk_copy(c, b).wait()
      do_chunk(c, bufs[b])

      @pl.when(c + 2 < NCHUNK)
      def _start_next():
        chunk_copy(c + 2, b).start()
    return 0
  lax.fori_loop(0, (NCHUNK - 1) // 2, pair_body, 0)

  chunk_copy(NCHUNK - 1, 0).wait()
  tail_cp = pltpu.make_async_copy(
      tail_hbm.at[pl.ds(r * TAIL_W, TAIL_W)], tbuf, sem1)
  tail_cp.start()
  do_chunk(NCHUNK - 1, buf0)
  tail_cp.wait()
  tacc = ninf_v
  for i in range(TAIL_VREGS):
    tacc = jnp.maximum(tacc, tbuf[pl.ds(i * 16, 16)])
  bmax[pl.ds(NBLK_FULL * 16, 16)] = tacc
  tpos = jnp.full((16,), NBLK_FULL, jnp.int32)
  plsc.store_scatter(bblk, [tpos],
                     jnp.broadcast_to(jnp.max(tacc), (16,)), mask=lane0)

  def key_body(i, _):
    m = lax.bitcast_convert_type(bblk[pl.ds(i * 16, 16)], jnp.int32)
    bkey[pl.ds(i * 16, 16)] = _fold_key(m)
    return 0
  lax.fori_loop(0, NBLK_PAD // 16, key_body, 0)

  def bit_body(b, prefix_u):
    cand_u = lax.bitwise_or(prefix_u, lax.shift_left(np.int32(1), 31 - b))
    cand_s = lax.bitwise_xor(cand_u, np.int32(-2**31))
    def cnt_body(i, acc):
      kv = bkey[pl.ds(i * 16, 16)]
      return acc + jnp.where(kv >= cand_s, np.int32(1), np.int32(0))
    acc = lax.fori_loop(0, NBLK_PAD // 16, cnt_body,
                        jnp.zeros((16,), jnp.int32))
    return jnp.where(jnp.sum(acc) >= TOPK, cand_u, prefix_u)
  v64u = lax.fori_loop(0, 32, bit_body, np.int32(0))
  v64k = lax.bitwise_xor(v64u, np.int32(-2**31))
  t = lax.bitcast_convert_type(_fold_key(v64k), jnp.float32)

  cc_ref[0] = np.int32(0)
  cc_ref[1] = np.int32(0)

  def append_vreg(x, gbase):
    msk = x >= t
    @pl.when(jnp.any(msk))
    def _append():
      mi = jnp.where(msk, np.int32(1), np.int32(0))
      cc = cc_ref[0]
      @pl.when(cc < CAP - 16)
      def _store():
        pos = plsc.cumsum(mi) + (cc - 1)
        plsc.store_scatter(cand_v, [pos], x, mask=msk)
        plsc.store_scatter(cand_i, [pos], gbase + iota, mask=msk)
      cc_ref[0] = cc + jnp.sum(mi)

  def qscan(i, _):
    ids = i * 16 + iota
    msk = (bblk[pl.ds(i * 16, 16)] >= t) & (ids < NBLK_FULL)
    @pl.when(jnp.any(msk))
    def _append_ids():
      mi = jnp.where(msk, np.int32(1), np.int32(0))
      qc = cc_ref[1]
      pos = plsc.cumsum(mi) + (qc - 1)
      plsc.store_scatter(qlist, [pos], ids, mask=msk)
      cc_ref[1] = qc + jnp.sum(mi)
    return 0
  lax.fori_loop(0, NBLK_PAD // 16, qscan, 0)
  qn = cc_ref[1]

  def qid_at(f):
    return plsc.load_gather(qlist, [jnp.broadcast_to(f, (16,))])[0]

  qbufs = (bbuf, bbuf2)
  def qcopy(f, b):
    return pltpu.make_async_copy(
        lp_hbm.at[pl.ds(2 * r, 2), pl.ds(qid_at(f) * BLK, BLK)],
        qbufs[b], sems[b])

  for b in range(2):
    @pl.when(b < qn)
    def _prime_q():
      qcopy(b, b).start()

  def qpair(k, _):
    for b in range(2):
      f = 2 * k + b
      @pl.when(f < qn)
      def _scan_q():
        qcopy(f, b).wait()
        bid = qid_at(f)
        def v_body(i, _):
          append_vreg(qbufs[b][0, pl.ds(i * 16, 16)], bid * BLK + i * 16)
          return 0
        lax.fori_loop(0, VPB, v_body, 0)
        @pl.when(f + 2 < qn)
        def _next_q():
          qcopy(f + 2, b).start()
    return 0
  lax.fori_loop(0, (qn + 1) // 2, qpair, 0)

  @pl.when(jnp.any(bmax[pl.ds(NBLK_FULL * 16, 16)] >= t))
  def _scan_tail():
    pltpu.sync_copy(tail_hbm.at[pl.ds(r * TAIL_W, TAIL_W)], tbuf)
    def v_body(i, _):
      append_vreg(tbuf[pl.ds(i * 16, 16)], TAIL_OFF + i * 16)
      return 0
    lax.fori_loop(0, TAIL_VREGS, v_body, 0)

  nv = (jnp.minimum(cc_ref[0], CAP) + 15) // 16

  def round_body(k, _):
    def mx(i, acc):
      return jnp.maximum(acc, cand_v[pl.ds(i * 16, 16)])
    m = jnp.max(lax.fori_loop(0, nv, mx, ninf_v))
    def mi(i, acc):
      hit = cand_v[pl.ds(i * 16, 16)] == m
      return jnp.minimum(acc, jnp.where(hit, cand_i[pl.ds(i * 16, 16)], IMAX))
    j = jnp.min(lax.fori_loop(0, nv, mi, jnp.full((16,), IMAX, jnp.int32)))
    def cl(i, _):
      hit = (cand_v[pl.ds(i * 16, 16)] == m) & (cand_i[pl.ds(i * 16, 16)] == j)
      cand_v[pl.ds(i * 16, 16)] = jnp.where(hit, NEG_INF,
                                            cand_v[pl.ds(i * 16, 16)])
      return 0
    lax.fori_loop(0, nv, cl, 0)
    kpos = jnp.broadcast_to(k, (16,)).astype(jnp.int32)
    plsc.store_scatter(topv, [kpos], jnp.broadcast_to(m, (16,)), mask=lane0)
    plsc.store_scatter(topi, [kpos], jnp.broadcast_to(j, (16,)), mask=lane0)
    return 0
  lax.fori_loop(0, TOPK, round_body, 0)

  pltpu.sync_copy(g_hbm.at[pl.ds(r * OUTW, OUTW)], gbuf)
  for i in range(OUTW // 16):
    ovec[pl.ds(i * 16, 16)] = jnp.zeros((16,), jnp.float32)
    oivec[pl.ds(i * 16, 16)] = jnp.zeros((16,), jnp.int32)
  for d in range(2):
    svecs = [topv[pl.ds(q * 16, 16)] + gbuf[pl.ds(d * TOPK + q * 16, 16)]
             for q in range(TOPK // 16)]
    m = jnp.max(jnp.maximum(jnp.maximum(svecs[0], svecs[1]),
                            jnp.maximum(svecs[2], svecs[3])))
    kacc = jnp.full((16,), IMAX, jnp.int32)
    for q in range(TOPK // 16):
      kacc = jnp.minimum(kacc, jnp.where(svecs[q] == m, iota + q * 16, IMAX))
    p = jnp.min(kacc)
    sel = jnp.broadcast_to(p, (16,))
    dpos = jnp.full((16,), d, jnp.int32)
    plsc.store_scatter(ovec, [dpos], plsc.load_gather(topv, [sel]),
                       mask=lane0)
    plsc.store_scatter(oivec, [dpos], plsc.load_gather(topi, [sel]),
                       mask=lane0)

  pltpu.sync_copy(ovec, outs_hbm.at[pl.ds(r * OUTW, OUTW)])
  pltpu.sync_copy(oivec, outi_hbm.at[pl.ds(r * OUTW, OUTW)])


@jax.jit
def _sc_topk_sample(lp2, tail_flat, g_flat):
  mesh = plsc.VectorSubcoreMesh(core_axis_name="c", subcore_axis_name="s")
  fn = pl.kernel(
      _row_body,
      out_type=[
          jax.ShapeDtypeStruct((NROW * OUTW,), jnp.float32),
          jax.ShapeDtypeStruct((NROW * OUTW,), jnp.int32),
      ],
      mesh=mesh,
      compiler_params=pltpu.CompilerParams(needs_layout_passes=False),
      scratch_types=[
          pltpu.VMEM((2, CHUNK), jnp.float32),
          pltpu.VMEM((2, CHUNK), jnp.float32),
          pltpu.VMEM((2, BLK), jnp.float32),
          pltpu.VMEM((2, BLK), jnp.float32),
          pltpu.VMEM((TAIL_W,), jnp.float32),
          pltpu.VMEM((NLM_PAD,), jnp.float32),
          pltpu.VMEM((NBLK_PAD,), jnp.float32),
          pltpu.VMEM((NBLK_PAD,), jnp.int32),
          pltpu.VMEM((NBLK_PAD,), jnp.int32),
          pltpu.VMEM((CAP,), jnp.float32),
          pltpu.VMEM((CAP,), jnp.int32),
          pltpu.VMEM((TOPK,), jnp.float32),
          pltpu.VMEM((TOPK,), jnp.int32),
          pltpu.VMEM((2 * TOPK,), jnp.float32),
          pltpu.VMEM((OUTW,), jnp.float32),
          pltpu.VMEM((OUTW,), jnp.int32),
          pltpu.SMEM((2,), jnp.int32),
          pltpu.SemaphoreType.DMA,
          pltpu.SemaphoreType.DMA,
      ],
  )
  return fn(lp2, tail_flat, g_flat)


def kernel(step, lprobs, scores):
  bsz, beam_size, _ = lprobs.shape
  g = jax.random.gumbel(jax.random.key(42), (beam_size, bsz, TOPK),
                        jnp.float32)
  gmat = jnp.zeros((bsz, OUTW), jnp.float32)
  gmat = gmat.at[:, :beam_size * TOPK].set(
      jnp.transpose(g, (1, 0, 2)).reshape(bsz, beam_size * TOPK))
  lp2 = lprobs.reshape(bsz * beam_size, VOCAB)
  tail_p = jnp.full((bsz, TAIL_W), NEG_INF, jnp.float32)
  tail_p = tail_p.at[:, :VOCAB - TAIL_OFF].set(lprobs[:, 0, TAIL_OFF:])
  outs, outi = _sc_topk_sample(lp2, tail_p.reshape(-1), gmat.reshape(-1))
  outs = outs.reshape(NROW, OUTW)
  outi = outi.reshape(NROW, OUTW)
  scores_buf = outs[:, :beam_size] + scores[:, :, 0] * step
  indices_buf = outi[:, :beam_size]
  beams_buf = jnp.full((bsz, beam_size), step, dtype=indices_buf.dtype)
  return scores_buf, indices_buf, beams_buf
